# Initial kernel scaffold; baseline (speedup 1.0000x reference)
import jax
import jax.numpy as jnp
from jax.experimental import pallas as pl


def kernel(x, edge_index, edge_attr, W, att_src, att_dst, W_e, att_edge, bias, W1, b1, W2, b2):
    raise NotImplementedError("write your pallas kernel here")


# speedup vs baseline: 12.6272x; 12.6272x over previous
import functools

import jax
import jax.numpy as jnp
from jax import lax
from jax.experimental import pallas as pl
from jax.experimental.pallas import tpu as pltpu
from jax.experimental.pallas import tpu_sc as plsc

N = 10000
NPAD = 10240
E = 320000
D = 128
H = 64
D_EDGE = 16
NPB = 100
HID = 256
OUT_DIM = 64
NC = 2
NS = 16
L = 16
SPAN = NPAD // NS
EPT1 = E // (NC * NS)
CH1 = 2000
EPT2 = E // NS
CH2 = 400
KBLK = 1280
KN = (NPB * D) // KBLK

_mesh = plsc.VectorSubcoreMesh(
    core_axis_name="c", subcore_axis_name="s", num_cores=NC, num_subcores=NS)


def _prep_body(x_ref, w_ref, att2_ref, xp_ref, asd_ref):
    xpb = lax.dot_general(x_ref[...], w_ref[...], (((1,), (0,)), ((), ())),
                          precision=lax.Precision.HIGHEST,
                          preferred_element_type=jnp.float32)
    xp_ref[0] = xpb[:, :H]
    xp_ref[1] = xpb[:, H:]
    asd_ref[...] = lax.dot_general(att2_ref[...], xpb,
                                   (((1,), (1,)), ((), ())),
                                   precision=lax.Precision.HIGHEST,
                                   preferred_element_type=jnp.float32)


def _prep(x_pad, W, att2):
    blk = 1280
    return pl.pallas_call(
        _prep_body,
        grid=(NPAD // blk,),
        in_specs=[
            pl.BlockSpec((blk, D), lambda i: (i, 0)),
            pl.BlockSpec((D, D), lambda i: (0, 0)),
            pl.BlockSpec((2, D), lambda i: (0, 0)),
        ],
        out_specs=[
            pl.BlockSpec((2, blk, H), lambda i: (0, i, 0)),
            pl.BlockSpec((2, blk), lambda i: (0, i)),
        ],
        out_shape=[
            jax.ShapeDtypeStruct((2, NPAD, H), jnp.float32),
            jax.ShapeDtypeStruct((2, NPAD), jnp.float32),
        ],
    )(x_pad, W, att2)


def _ae_body(attr_ref, we_ref, atte_ref, ae_ref):
    v = lax.dot_general(we_ref[...], atte_ref[...], (((1,), (0,)), ((), ())),
                        precision=lax.Precision.HIGHEST,
                        preferred_element_type=jnp.float32)
    vrep = jnp.concatenate([v] * 8, axis=0)
    vrep = jnp.broadcast_to(vrep, (D, 8))
    ki = lax.broadcasted_iota(jnp.int32, (D, 8), 0)
    ji = lax.broadcasted_iota(jnp.int32, (D, 8), 1)
    vdiag = jnp.where((ki // D_EDGE) == ji, vrep, 0.0)
    ae_ref[...] = lax.dot_general(attr_ref[...], vdiag,
                                  (((1,), (0,)), ((), ())),
                                  precision=lax.Precision.HIGHEST,
                                  preferred_element_type=jnp.float32)


def _ae(edge_attr8, W_e, att_edge_col):
    blk = 5000
    rows8 = E // 8
    return pl.pallas_call(
        _ae_body,
        grid=(rows8 // blk,),
        in_specs=[
            pl.BlockSpec((blk, D), lambda i: (i, 0)),
            pl.BlockSpec((D_EDGE, D), lambda i: (0, 0)),
            pl.BlockSpec((D, 1), lambda i: (0, 0)),
        ],
        out_specs=pl.BlockSpec((blk, 8), lambda i: (i, 0)),
        out_shape=jax.ShapeDtypeStruct((rows8, 8), jnp.float32),
    )(edge_attr8, W_e, att_edge_col)


@functools.partial(
    pl.kernel,
    out_type=[
        jax.ShapeDtypeStruct((E,), jnp.float32),
        jax.ShapeDtypeStruct((NPAD,), jnp.float32),
        jax.ShapeDtypeStruct((NPAD,), jnp.float32),
    ],
    mesh=_mesh,
    scratch_types=[
        pltpu.VMEM((NPAD,), jnp.float32),
        pltpu.VMEM((NPAD,), jnp.float32),
        pltpu.VMEM((NPAD,), jnp.float32),
        pltpu.VMEM((CH1,), jnp.int32),
        pltpu.VMEM((CH1,), jnp.int32),
        pltpu.VMEM((CH1,), jnp.float32),
        pltpu.VMEM((CH1,), jnp.float32),
        pltpu.VMEM_SHARED((NS, NPAD), jnp.float32),
        pltpu.VMEM((SPAN,), jnp.float32),
        pltpu.VMEM((SPAN,), jnp.float32),
    ],
    compiler_params=pltpu.CompilerParams(needs_layout_passes=False, use_tc_tiling_on_sc=False),
)
def _pass1(a_s_hbm, a_d_hbm, srcE, dstE, ae, ex_out, den0_out, den1_out,
           a_s_v, a_d_v, den_v, src_v, dst_v, ae_v, ex_v,
           den_sh, racc_v, rrow_v):
    c = lax.axis_index("c")
    s = lax.axis_index("s")
    wid = c * NS + s
    pltpu.sync_copy(a_s_hbm, a_s_v)
    pltpu.sync_copy(a_d_hbm, a_d_v)

    def zbody(i, carry):
        den_v[pl.ds(i * L, L)] = jnp.zeros((L,), jnp.float32)
        return carry
    lax.fori_loop(0, NPAD // L, zbody, 0)

    base0 = wid * EPT1
    for k in range(EPT1 // CH1):
        base = base0 + k * CH1
        pltpu.sync_copy(srcE.at[pl.ds(base, CH1)], src_v)
        pltpu.sync_copy(dstE.at[pl.ds(base, CH1)], dst_v)
        pltpu.sync_copy(ae.at[pl.ds(base, CH1)], ae_v)

        def ebody(j, carry):
            si = src_v[pl.ds(j * L, L)]
            di = dst_v[pl.ds(j * L, L)]
            av = (plsc.load_gather(a_s_v, [si])
                  + plsc.load_gather(a_d_v, [di])
                  + ae_v[pl.ds(j * L, L)])
            av = jnp.where(av >= 0.0, av, av * jnp.float32(0.2))
            ev = jnp.exp(av)
            ex_v[pl.ds(j * L, L)] = ev
            plsc.addupdate_scatter(den_v, [di], ev)
            return carry
        lax.fori_loop(0, CH1 // L, ebody, 0)
        pltpu.sync_copy(ex_v, ex_out.at[pl.ds(base, CH1)])

    pltpu.sync_copy(den_v, den_sh.at[s])
    plsc.subcore_barrier()
    pltpu.sync_copy(den_sh.at[0, pl.ds(s * SPAN, SPAN)], racc_v)
    for r in range(1, NS):
        pltpu.sync_copy(den_sh.at[r, pl.ds(s * SPAN, SPAN)], rrow_v)

        def abody(i, carry):
            racc_v[pl.ds(i * L, L)] = (racc_v[pl.ds(i * L, L)]
                                       + rrow_v[pl.ds(i * L, L)])
            return carry
        lax.fori_loop(0, SPAN // L, abody, 0)

    @pl.when(c == 0)
    def _():
        pltpu.sync_copy(racc_v, den0_out.at[pl.ds(s * SPAN, SPAN)])

    @pl.when(c == 1)
    def _():
        pltpu.sync_copy(racc_v, den1_out.at[pl.ds(s * SPAN, SPAN)])


@functools.partial(
    pl.kernel,
    out_type=jax.ShapeDtypeStruct((NC, NPAD, H), jnp.float32),
    mesh=_mesh,
    scratch_types=[
        pltpu.VMEM_SHARED((NPAD, H), jnp.float32),
        pltpu.VMEM_SHARED((NPAD, H), jnp.float32),
        pltpu.VMEM((CH2,), jnp.int32),
        pltpu.VMEM((CH2,), jnp.int32),
        pltpu.VMEM((CH2,), jnp.float32),
        pltpu.VMEM((CH2, H), jnp.float32),
        pltpu.VMEM((SPAN,), jnp.float32),
        pltpu.VMEM((SPAN,), jnp.float32),
        pltpu.VMEM((D,), jnp.float32),
        pltpu.SemaphoreType.DMA,
    ],
    compiler_params=pltpu.CompilerParams(needs_layout_passes=False, use_tc_tiling_on_sc=False),
)
def _pass2(xp_hbm, srcE, dstE, ex_hbm, den0_hbm, den1_hbm, bias_hbm, nout,
           xp_sh, acc_sh, src_v, dst_v, ex_v, rows_v,
           den0_v, den1_v, bias_v, sem):
    c = lax.axis_index("c")
    s = lax.axis_index("s")
    rows = pl.ds(s * SPAN, SPAN)
    HS = SPAN // 2

    @pl.when(c == 0)
    def _():
        pltpu.sync_copy(xp_hbm.at[0].at[rows], xp_sh.at[rows])

    @pl.when(c == 1)
    def _():
        pltpu.sync_copy(xp_hbm.at[1].at[rows], xp_sh.at[rows])

    def zrow(t, carry):
        for j in range(H // L):
            rows_v[t, pl.ds(j * L, L)] = jnp.zeros((L,), jnp.float32)
        return carry
    lax.fori_loop(0, HS, zrow, 0)
    for i in range(SPAN // HS):
        pltpu.sync_copy(rows_v.at[pl.ds(0, HS)],
                        acc_sh.at[pl.ds(s * SPAN + i * HS, HS)])
    plsc.subcore_barrier()

    base0 = s * EPT2

    def chunk(k, carry):
        base = base0 + k * CH2
        pltpu.sync_copy(srcE.at[pl.ds(base, CH2)], src_v)
        pltpu.sync_copy(dstE.at[pl.ds(base, CH2)], dst_v)
        pltpu.sync_copy(ex_hbm.at[pl.ds(base, CH2)], ex_v)
        pltpu.async_copy(xp_sh.at[src_v], rows_v, sem).wait()

        def scale(t, carry2):
            e16 = ex_v[pl.ds(t * L, L)]
            for i in range(L):
                r = t * L + i
                e = e16[i]
                for j in range(H // L):
                    rows_v[r, pl.ds(j * L, L)] = \
                        rows_v[r, pl.ds(j * L, L)] * e
            return carry2
        lax.fori_loop(0, CH2 // L, scale, 0)
        pltpu.sync_copy(rows_v, acc_sh.at[dst_v], add=True)
        return carry
    lax.fori_loop(0, EPT2 // CH2, chunk, 0)

    plsc.subcore_barrier()

    pltpu.sync_copy(den0_hbm.at[rows], den0_v)
    pltpu.sync_copy(den1_hbm.at[rows], den1_v)
    pltpu.sync_copy(bias_hbm, bias_v)
    bvs = [bias_v[pl.ds(c * H + j * L, L)] for j in range(H // L)]

    for i in range(SPAN // HS):
        half = pl.ds(s * SPAN + i * HS, HS)
        pltpu.sync_copy(acc_sh.at[half], rows_v.at[pl.ds(0, HS)])

        def fin(t, carry):
            d16 = (den0_v[pl.ds(i * HS + t * L, L)]
                   + den1_v[pl.ds(i * HS + t * L, L)]
                   + jnp.float32(1e-16))
            rc16 = jnp.float32(1.0) / d16
            for u in range(L):
                r = t * L + u
                rcp = rc16[u]
                for j in range(H // L):
                    rows_v[r, pl.ds(j * L, L)] = \
                        rows_v[r, pl.ds(j * L, L)] * rcp + bvs[j]
            return carry
        lax.fori_loop(0, HS // L, fin, 0)

        @pl.when(c == 0)
        def _():
            pltpu.sync_copy(rows_v.at[pl.ds(0, HS)], nout.at[0].at[half])

        @pl.when(c == 1)
        def _():
            pltpu.sync_copy(rows_v.at[pl.ds(0, HS)], nout.at[1].at[half])


def _fc_body(g_ref, w1_ref, b1_ref, w2_ref, b2_ref, o_ref, acc_ref):
    k = pl.program_id(0)

    @pl.when(k == 0)
    def _():
        acc_ref[...] = jnp.zeros_like(acc_ref)

    acc_ref[...] += lax.dot_general(g_ref[...], w1_ref[...],
                                    (((1,), (0,)), ((), ())),
                                    precision=lax.Precision.HIGHEST,
                                    preferred_element_type=jnp.float32)

    @pl.when(k == KN - 1)
    def _():
        h = jnp.maximum(acc_ref[...] + b1_ref[...], 0.0)
        o_ref[...] = lax.dot_general(h, w2_ref[...],
                                     (((1,), (0,)), ((), ())),
                                     precision=lax.Precision.HIGHEST,
                                     preferred_element_type=jnp.float32) \
            + b2_ref[...]


def _fc(g, W1, b1_row, W2, b2_row):
    nb = N // NPB
    return pl.pallas_call(
        _fc_body,
        grid=(KN,),
        in_specs=[
            pl.BlockSpec((nb, KBLK), lambda k: (0, k)),
            pl.BlockSpec((KBLK, HID), lambda k: (k, 0)),
            pl.BlockSpec((1, HID), lambda k: (0, 0)),
            pl.BlockSpec((HID, OUT_DIM), lambda k: (0, 0)),
            pl.BlockSpec((1, OUT_DIM), lambda k: (0, 0)),
        ],
        out_specs=pl.BlockSpec((nb, OUT_DIM), lambda k: (0, 0)),
        out_shape=jax.ShapeDtypeStruct((nb, OUT_DIM), jnp.float32),
        scratch_shapes=[pltpu.VMEM((nb, HID), jnp.float32)],
    )(g, W1, b1_row, W2, b2_row)


def kernel(x, edge_index, edge_attr, W, att_src, att_dst, W_e, att_edge,
           bias, W1, b1, W2, b2):
    x_pad = jnp.pad(x, ((0, NPAD - N), (0, 0)))
    att2 = jnp.stack([att_src, att_dst])
    xp, asd = _prep(x_pad, W, att2)
    ae = _ae(edge_attr.reshape(E // 8, D), W_e,
             att_edge.reshape(D, 1)).reshape(E)
    srcE = edge_index[0]
    dstE = edge_index[1]
    a_s_arr = asd[0]
    a_d_arr = asd[1]
    ex, den0, den1 = _pass1(a_s_arr, a_d_arr, srcE, dstE, ae)
    nout = _pass2(xp, srcE, dstE, ex, den0, den1, bias)
    g = jnp.concatenate([nout[0, :N], nout[1, :N]], axis=1)
    g = g.reshape(N // NPB, NPB * D)
    return _fc(g, W1, b1.reshape(1, HID), W2, b2.reshape(1, OUT_DIM))


# speedup vs baseline: 14.1078x; 1.1173x over previous
import functools

import jax
import jax.numpy as jnp
from jax import lax
from jax.experimental import pallas as pl
from jax.experimental.pallas import tpu as pltpu
from jax.experimental.pallas import tpu_sc as plsc

N = 10000
NPAD = 10240
E = 320000
D = 128
H = 64
D_EDGE = 16
NPB = 100
HID = 256
OUT_DIM = 64
NC = 2
NS = 16
L = 16
SPAN = NPAD // NS
EPT1 = E // (NC * NS)
CH1 = 2000
EPT2 = E // NS
CH2 = 160
NCH = EPT2 // CH2
NPAIR = NCH // 2
PKW = 3 * CH2
KBLK = 1280
KN = (NPB * D) // KBLK

_mesh = plsc.VectorSubcoreMesh(
    core_axis_name="c", subcore_axis_name="s", num_cores=NC, num_subcores=NS)


def _prep_body(x_ref, w_ref, att2_ref, xp_ref, asd_ref):
    xpb = lax.dot_general(x_ref[...], w_ref[...], (((1,), (0,)), ((), ())),
                          precision=lax.Precision.HIGHEST,
                          preferred_element_type=jnp.float32)
    xp_ref[0] = xpb[:, :H]
    xp_ref[1] = xpb[:, H:]
    asd_ref[...] = lax.dot_general(att2_ref[...], xpb,
                                   (((1,), (1,)), ((), ())),
                                   precision=lax.Precision.HIGHEST,
                                   preferred_element_type=jnp.float32)


def _prep(x_pad, W, att2):
    blk = 1280
    return pl.pallas_call(
        _prep_body,
        grid=(NPAD // blk,),
        in_specs=[
            pl.BlockSpec((blk, D), lambda i: (i, 0)),
            pl.BlockSpec((D, D), lambda i: (0, 0)),
            pl.BlockSpec((2, D), lambda i: (0, 0)),
        ],
        out_specs=[
            pl.BlockSpec((2, blk, H), lambda i: (0, i, 0)),
            pl.BlockSpec((2, blk), lambda i: (0, i)),
        ],
        out_shape=[
            jax.ShapeDtypeStruct((2, NPAD, H), jnp.float32),
            jax.ShapeDtypeStruct((2, NPAD), jnp.float32),
        ],
    )(x_pad, W, att2)


def _ae_body(attr_ref, we_ref, atte_ref, ae_ref):
    v = lax.dot_general(we_ref[...], atte_ref[...], (((1,), (0,)), ((), ())),
                        precision=lax.Precision.HIGHEST,
                        preferred_element_type=jnp.float32)
    vrep = jnp.concatenate([v] * 8, axis=0)
    vrep = jnp.broadcast_to(vrep, (D, 8))
    ki = lax.broadcasted_iota(jnp.int32, (D, 8), 0)
    ji = lax.broadcasted_iota(jnp.int32, (D, 8), 1)
    vdiag = jnp.where((ki // D_EDGE) == ji, vrep, 0.0)
    ae_ref[...] = lax.dot_general(attr_ref[...], vdiag,
                                  (((1,), (0,)), ((), ())),
                                  precision=lax.Precision.HIGHEST,
                                  preferred_element_type=jnp.float32)


def _ae(edge_attr8, W_e, att_edge_col):
    blk = 5000
    rows8 = E // 8
    return pl.pallas_call(
        _ae_body,
        grid=(rows8 // blk,),
        in_specs=[
            pl.BlockSpec((blk, D), lambda i: (i, 0)),
            pl.BlockSpec((D_EDGE, D), lambda i: (0, 0)),
            pl.BlockSpec((D, 1), lambda i: (0, 0)),
        ],
        out_specs=pl.BlockSpec((blk, 8), lambda i: (i, 0)),
        out_shape=jax.ShapeDtypeStruct((rows8, 8), jnp.float32),
    )(edge_attr8, W_e, att_edge_col)


@functools.partial(
    pl.kernel,
    out_type=[
        jax.ShapeDtypeStruct((E,), jnp.float32),
        jax.ShapeDtypeStruct((NPAD,), jnp.float32),
        jax.ShapeDtypeStruct((NPAD,), jnp.float32),
    ],
    mesh=_mesh,
    scratch_types=[
        pltpu.VMEM((NPAD,), jnp.float32),
        pltpu.VMEM((NPAD,), jnp.float32),
        pltpu.VMEM((NPAD,), jnp.float32),
        pltpu.VMEM((CH1,), jnp.int32),
        pltpu.VMEM((CH1,), jnp.int32),
        pltpu.VMEM((CH1,), jnp.float32),
        pltpu.VMEM((CH1,), jnp.float32),
        pltpu.VMEM_SHARED((NS, NPAD), jnp.float32),
        pltpu.VMEM((SPAN,), jnp.float32),
        pltpu.VMEM((SPAN,), jnp.float32),
    ],
    compiler_params=pltpu.CompilerParams(needs_layout_passes=False, use_tc_tiling_on_sc=False),
)
def _pass1(a_s_hbm, a_d_hbm, srcE, dstE, ae, ex_out, den0_out, den1_out,
           a_s_v, a_d_v, den_v, src_v, dst_v, ae_v, ex_v,
           den_sh, racc_v, rrow_v):
    c = lax.axis_index("c")
    s = lax.axis_index("s")
    wid = c * NS + s
    pltpu.sync_copy(a_s_hbm, a_s_v)
    pltpu.sync_copy(a_d_hbm, a_d_v)

    def zbody(i, carry):
        den_v[pl.ds(i * L, L)] = jnp.zeros((L,), jnp.float32)
        return carry
    lax.fori_loop(0, NPAD // L, zbody, 0)

    base0 = wid * EPT1
    for k in range(EPT1 // CH1):
        base = base0 + k * CH1
        pltpu.sync_copy(srcE.at[pl.ds(base, CH1)], src_v)
        pltpu.sync_copy(dstE.at[pl.ds(base, CH1)], dst_v)
        pltpu.sync_copy(ae.at[pl.ds(base, CH1)], ae_v)

        def ebody(j, carry):
            si = src_v[pl.ds(j * L, L)]
            di = dst_v[pl.ds(j * L, L)]
            av = (plsc.load_gather(a_s_v, [si])
                  + plsc.load_gather(a_d_v, [di])
                  + ae_v[pl.ds(j * L, L)])
            av = jnp.where(av >= 0.0, av, av * jnp.float32(0.2))
            ev = jnp.exp(av)
            ex_v[pl.ds(j * L, L)] = ev
            plsc.addupdate_scatter(den_v, [di], ev)
            return carry
        lax.fori_loop(0, CH1 // L, ebody, 0)
        pltpu.sync_copy(ex_v, ex_out.at[pl.ds(base, CH1)])

    pltpu.sync_copy(den_v, den_sh.at[s])
    plsc.subcore_barrier()
    pltpu.sync_copy(den_sh.at[0, pl.ds(s * SPAN, SPAN)], racc_v)
    for r in range(1, NS):
        pltpu.sync_copy(den_sh.at[r, pl.ds(s * SPAN, SPAN)], rrow_v)

        def abody(i, carry):
            racc_v[pl.ds(i * L, L)] = (racc_v[pl.ds(i * L, L)]
                                       + rrow_v[pl.ds(i * L, L)])
            return carry
        lax.fori_loop(0, SPAN // L, abody, 0)

    @pl.when(c == 0)
    def _():
        pltpu.sync_copy(racc_v, den0_out.at[pl.ds(s * SPAN, SPAN)])

    @pl.when(c == 1)
    def _():
        pltpu.sync_copy(racc_v, den1_out.at[pl.ds(s * SPAN, SPAN)])


@functools.partial(
    pl.kernel,
    out_type=jax.ShapeDtypeStruct((NC, NPAD, H), jnp.float32),
    mesh=_mesh,
    scratch_types=[
        pltpu.VMEM_SHARED((NPAD, H), jnp.float32),
        pltpu.VMEM_SHARED((NPAD, H), jnp.float32),
        pltpu.VMEM((PKW,), jnp.int32),
        pltpu.VMEM((PKW,), jnp.int32),
        pltpu.VMEM((CH2, H), jnp.float32),
        pltpu.VMEM((CH2, H), jnp.float32),
        pltpu.VMEM((SPAN,), jnp.float32),
        pltpu.VMEM((SPAN,), jnp.float32),
        pltpu.VMEM((D,), jnp.float32),
        pltpu.SemaphoreType.DMA,
        pltpu.SemaphoreType.DMA,
        pltpu.SemaphoreType.DMA,
        pltpu.SemaphoreType.DMA,
        pltpu.SemaphoreType.DMA,
        pltpu.SemaphoreType.DMA,
    ],
    compiler_params=pltpu.CompilerParams(needs_layout_passes=False, use_tc_tiling_on_sc=False),
)
def _pass2(xp_hbm, pk_hbm, den0_hbm, den1_hbm, bias_hbm, nout,
           xp_sh, acc_sh, pkA, pkB, rowsA, rowsB,
           den0_v, den1_v, bias_v, siA, siB, sgA, sgB, ssA, ssB):
    c = lax.axis_index("c")
    s = lax.axis_index("s")
    rows = pl.ds(s * SPAN, SPAN)

    @pl.when(c == 0)
    def _():
        pltpu.sync_copy(xp_hbm.at[0].at[rows], xp_sh.at[rows])

    @pl.when(c == 1)
    def _():
        pltpu.sync_copy(xp_hbm.at[1].at[rows], xp_sh.at[rows])

    def zrow(t, carry):
        for j in range(H // L):
            rowsA[t, pl.ds(j * L, L)] = jnp.zeros((L,), jnp.float32)
        return carry
    lax.fori_loop(0, CH2, zrow, 0)
    for i in range(SPAN // CH2):
        pltpu.sync_copy(rowsA, acc_sh.at[pl.ds(s * SPAN + i * CH2, CH2)])
    plsc.subcore_barrier()

    def load_idx(k, pkX, sem):
        g = s * NCH + k
        pltpu.async_copy(pk_hbm.at[pl.ds(g * PKW, PKW)], pkX, sem)

    def wait_idx(pkX, sem):
        pltpu.make_async_copy(pk_hbm.at[pl.ds(0, PKW)], pkX, sem).wait()

    def gather(pkX, rowsX, sem):
        pltpu.async_copy(xp_sh.at[pkX.at[pl.ds(0, CH2)]], rowsX, sem)

    def wait_gather(pkX, rowsX, sem):
        pltpu.make_async_copy(
            xp_sh.at[pkX.at[pl.ds(0, CH2)]], rowsX, sem).wait()

    def scatter(rowsX, pkX, sem):
        pltpu.async_copy(rowsX, acc_sh.at[pkX.at[pl.ds(CH2, CH2)]], sem,
                         add=True)

    def wait_scatter(rowsX, pkX, sem):
        pltpu.make_async_copy(
            rowsX, acc_sh.at[pkX.at[pl.ds(CH2, CH2)]], sem).wait()

    def scale(pkX, rowsX):
        def sbody(t, carry):
            e16 = plsc.bitcast(pkX[pl.ds(2 * CH2 + t * L, L)], jnp.float32)
            for u in range(L):
                r = t * L + u
                e = e16[u]
                for j in range(H // L):
                    rowsX[r, pl.ds(j * L, L)] = \
                        rowsX[r, pl.ds(j * L, L)] * e
            return carry
        lax.fori_loop(0, CH2 // L, sbody, 0)

    load_idx(0, pkA, siA)
    wait_idx(pkA, siA)
    gather(pkA, rowsA, sgA)
    load_idx(1, pkB, siB)

    def pair(p, carry):
        wait_idx(pkB, siB)

        @pl.when(p > 0)
        def _():
            wait_scatter(rowsB, pkB, ssB)
        gather(pkB, rowsB, sgB)
        wait_gather(pkA, rowsA, sgA)
        scale(pkA, rowsA)
        scatter(rowsA, pkA, ssA)
        wait_gather(pkB, rowsB, sgB)
        scale(pkB, rowsB)
        scatter(rowsB, pkB, ssB)

        @pl.when(p < NPAIR - 1)
        def _():
            wait_scatter(rowsA, pkA, ssA)
            load_idx(2 * p + 2, pkA, siA)
            wait_idx(pkA, siA)
            gather(pkA, rowsA, sgA)
            load_idx(2 * p + 3, pkB, siB)
        return carry
    lax.fori_loop(0, NPAIR, pair, 0)

    wait_scatter(rowsA, pkA, ssA)
    load_idx(NCH - 1, pkA, siA)
    wait_idx(pkA, siA)
    gather(pkA, rowsA, sgA)
    wait_gather(pkA, rowsA, sgA)
    scale(pkA, rowsA)
    scatter(rowsA, pkA, ssA)
    wait_scatter(rowsA, pkA, ssA)
    wait_scatter(rowsB, pkB, ssB)

    plsc.subcore_barrier()

    pltpu.sync_copy(den0_hbm.at[rows], den0_v)
    pltpu.sync_copy(den1_hbm.at[rows], den1_v)
    pltpu.sync_copy(bias_hbm, bias_v)
    bvs = [bias_v[pl.ds(c * H + j * L, L)] for j in range(H // L)]

    for i in range(SPAN // CH2):
        quarter = pl.ds(s * SPAN + i * CH2, CH2)
        pltpu.sync_copy(acc_sh.at[quarter], rowsA)

        def fin(t, carry):
            d16 = (den0_v[pl.ds(i * CH2 + t * L, L)]
                   + den1_v[pl.ds(i * CH2 + t * L, L)]
                   + jnp.float32(1e-16))
            rc16 = jnp.float32(1.0) / d16
            for u in range(L):
                r = t * L + u
                rcp = rc16[u]
                for j in range(H // L):
                    rowsA[r, pl.ds(j * L, L)] = \
                        rowsA[r, pl.ds(j * L, L)] * rcp + bvs[j]
            return carry
        lax.fori_loop(0, CH2 // L, fin, 0)

        @pl.when(c == 0)
        def _():
            pltpu.sync_copy(rowsA, nout.at[0].at[quarter])

        @pl.when(c == 1)
        def _():
            pltpu.sync_copy(rowsA, nout.at[1].at[quarter])


def _fc_body(g_ref, w1_ref, b1_ref, w2_ref, b2_ref, o_ref, acc_ref):
    k = pl.program_id(0)

    @pl.when(k == 0)
    def _():
        acc_ref[...] = jnp.zeros_like(acc_ref)

    acc_ref[...] += lax.dot_general(g_ref[...], w1_ref[...],
                                    (((1,), (0,)), ((), ())),
                                    precision=lax.Precision.HIGHEST,
                                    preferred_element_type=jnp.float32)

    @pl.when(k == KN - 1)
    def _():
        h = jnp.maximum(acc_ref[...] + b1_ref[...], 0.0)
        o_ref[...] = lax.dot_general(h, w2_ref[...],
                                     (((1,), (0,)), ((), ())),
                                     precision=lax.Precision.HIGHEST,
                                     preferred_element_type=jnp.float32) \
            + b2_ref[...]


def _fc(g, W1, b1_row, W2, b2_row):
    nb = N // NPB
    return pl.pallas_call(
        _fc_body,
        grid=(KN,),
        in_specs=[
            pl.BlockSpec((nb, KBLK), lambda k: (0, k)),
            pl.BlockSpec((KBLK, HID), lambda k: (k, 0)),
            pl.BlockSpec((1, HID), lambda k: (0, 0)),
            pl.BlockSpec((HID, OUT_DIM), lambda k: (0, 0)),
            pl.BlockSpec((1, OUT_DIM), lambda k: (0, 0)),
        ],
        out_specs=pl.BlockSpec((nb, OUT_DIM), lambda k: (0, 0)),
        out_shape=jax.ShapeDtypeStruct((nb, OUT_DIM), jnp.float32),
        scratch_shapes=[pltpu.VMEM((nb, HID), jnp.float32)],
    )(g, W1, b1_row, W2, b2_row)


def kernel(x, edge_index, edge_attr, W, att_src, att_dst, W_e, att_edge,
           bias, W1, b1, W2, b2):
    x_pad = jnp.pad(x, ((0, NPAD - N), (0, 0)))
    att2 = jnp.stack([att_src, att_dst])
    xp, asd = _prep(x_pad, W, att2)
    ae = _ae(edge_attr.reshape(E // 8, D), W_e,
             att_edge.reshape(D, 1)).reshape(E)
    srcE = edge_index[0]
    dstE = edge_index[1]
    a_s_arr = asd[0]
    a_d_arr = asd[1]
    ex, den0, den1 = _pass1(a_s_arr, a_d_arr, srcE, dstE, ae)
    ex_i = lax.bitcast_convert_type(ex, jnp.int32)
    ncg = E // CH2
    pk = jnp.concatenate([srcE.reshape(ncg, 1, CH2),
                          dstE.reshape(ncg, 1, CH2),
                          ex_i.reshape(ncg, 1, CH2)], axis=1).reshape(E * 3)
    nout = _pass2(xp, pk, den0, den1, bias)
    g = jnp.concatenate([nout[0, :N], nout[1, :N]], axis=1)
    g = g.reshape(N // NPB, NPB * D)
    return _fc(g, W1, b1.reshape(1, HID), W2, b2.reshape(1, OUT_DIM))


# speedup vs baseline: 14.3465x; 1.0169x over previous
import functools

import jax
import jax.numpy as jnp
from jax import lax
from jax.experimental import pallas as pl
from jax.experimental.pallas import tpu as pltpu
from jax.experimental.pallas import tpu_sc as plsc

N = 10000
NPAD = 10240
E = 320000
D = 128
H = 64
D_EDGE = 16
NPB = 100
HID = 256
OUT_DIM = 64
NC = 2
NS = 16
L = 16
SPAN = NPAD // NS
EPT1 = E // (NC * NS)
CH1 = 2000
EPT2 = E // NS
CH2 = 400
NCH = EPT2 // CH2
NPAIR = NCH // 2
PKW = 3 * CH2
NGRP = E // CH2
GPT1 = CH1 // CH2
KBLK = 1280
KN = (NPB * D) // KBLK

_mesh = plsc.VectorSubcoreMesh(
    core_axis_name="c", subcore_axis_name="s", num_cores=NC, num_subcores=NS)


def _prep_body(x_ref, w_ref, att2_ref, xp_ref, asd_ref):
    xpb = lax.dot_general(x_ref[...], w_ref[...], (((1,), (0,)), ((), ())),
                          precision=lax.Precision.HIGHEST,
                          preferred_element_type=jnp.float32)
    xp_ref[0] = xpb[:, :H]
    xp_ref[1] = xpb[:, H:]
    asd_ref[...] = lax.dot_general(att2_ref[...], xpb,
                                   (((1,), (1,)), ((), ())),
                                   precision=lax.Precision.HIGHEST,
                                   preferred_element_type=jnp.float32)


def _prep(x_pad, W, att2):
    blk = 1280
    return pl.pallas_call(
        _prep_body,
        grid=(NPAD // blk,),
        in_specs=[
            pl.BlockSpec((blk, D), lambda i: (i, 0)),
            pl.BlockSpec((D, D), lambda i: (0, 0)),
            pl.BlockSpec((2, D), lambda i: (0, 0)),
        ],
        out_specs=[
            pl.BlockSpec((2, blk, H), lambda i: (0, i, 0)),
            pl.BlockSpec((2, blk), lambda i: (0, i)),
        ],
        out_shape=[
            jax.ShapeDtypeStruct((2, NPAD, H), jnp.float32),
            jax.ShapeDtypeStruct((2, NPAD), jnp.float32),
        ],
    )(x_pad, W, att2)


def _ae_body(attr_ref, we_ref, atte_ref, ae_ref):
    v = lax.dot_general(we_ref[...], atte_ref[...], (((1,), (0,)), ((), ())),
                        precision=lax.Precision.HIGHEST,
                        preferred_element_type=jnp.float32)
    vrep = jnp.concatenate([v] * 8, axis=0)
    vrep = jnp.broadcast_to(vrep, (D, 8))
    ki = lax.broadcasted_iota(jnp.int32, (D, 8), 0)
    ji = lax.broadcasted_iota(jnp.int32, (D, 8), 1)
    vdiag = jnp.where((ki // D_EDGE) == ji, vrep, 0.0)
    ae_ref[...] = lax.dot_general(attr_ref[...], vdiag,
                                  (((1,), (0,)), ((), ())),
                                  precision=lax.Precision.HIGHEST,
                                  preferred_element_type=jnp.float32)


def _ae(edge_attr8, W_e, att_edge_col):
    blk = 5000
    rows8 = E // 8
    return pl.pallas_call(
        _ae_body,
        grid=(rows8 // blk,),
        in_specs=[
            pl.BlockSpec((blk, D), lambda i: (i, 0)),
            pl.BlockSpec((D_EDGE, D), lambda i: (0, 0)),
            pl.BlockSpec((D, 1), lambda i: (0, 0)),
        ],
        out_specs=pl.BlockSpec((blk, 8), lambda i: (i, 0)),
        out_shape=jax.ShapeDtypeStruct((rows8, 8), jnp.float32),
    )(edge_attr8, W_e, att_edge_col)


@functools.partial(
    pl.kernel,
    out_type=[
        jax.ShapeDtypeStruct((E * 3,), jnp.int32),
        jax.ShapeDtypeStruct((NPAD,), jnp.float32),
        jax.ShapeDtypeStruct((NPAD,), jnp.float32),
    ],
    mesh=_mesh,
    scratch_types=[
        pltpu.VMEM((NPAD,), jnp.float32),
        pltpu.VMEM((NPAD,), jnp.float32),
        pltpu.VMEM((NPAD,), jnp.float32),
        pltpu.VMEM((GPT1 * PKW,), jnp.int32),
        pltpu.VMEM((CH1,), jnp.float32),
        pltpu.VMEM_SHARED((NS, NPAD), jnp.float32),
        pltpu.VMEM((SPAN,), jnp.float32),
        pltpu.VMEM((SPAN,), jnp.float32),
    ],
    compiler_params=pltpu.CompilerParams(needs_layout_passes=False, use_tc_tiling_on_sc=False),
)
def _pass1(a_s_hbm, a_d_hbm, pk_sd, ae, pk_out, den0_out, den1_out,
           a_s_v, a_d_v, den_v, pkv, ae_v,
           den_sh, racc_v, rrow_v):
    c = lax.axis_index("c")
    s = lax.axis_index("s")
    wid = c * NS + s
    pltpu.sync_copy(a_s_hbm, a_s_v)
    pltpu.sync_copy(a_d_hbm, a_d_v)

    def zbody(i, carry):
        den_v[pl.ds(i * L, L)] = jnp.zeros((L,), jnp.float32)
        return carry
    lax.fori_loop(0, NPAD // L, zbody, 0)

    for m in range(EPT1 // CH1):
        woff = (wid * EPT1 + m * CH1) * 3
        pltpu.sync_copy(pk_sd.at[pl.ds(woff, GPT1 * PKW)], pkv)
        pltpu.sync_copy(ae.at[pl.ds(wid * EPT1 + m * CH1, CH1)], ae_v)

        for grp in range(GPT1):
            goff = grp * PKW

            def ebody(j, carry):
                si = pkv[pl.ds(goff + j * L, L)]
                di = pkv[pl.ds(goff + CH2 + j * L, L)]
                av = (plsc.load_gather(a_s_v, [si])
                      + plsc.load_gather(a_d_v, [di])
                      + ae_v[pl.ds(grp * CH2 + j * L, L)])
                av = jnp.where(av >= 0.0, av, av * jnp.float32(0.2))
                ev = jnp.exp(av)
                pkv[pl.ds(goff + 2 * CH2 + j * L, L)] = \
                    plsc.bitcast(ev, jnp.int32)
                plsc.addupdate_scatter(den_v, [di], ev)
                return carry
            lax.fori_loop(0, CH2 // L, ebody, 0)
        pltpu.sync_copy(pkv, pk_out.at[pl.ds(woff, GPT1 * PKW)])

    pltpu.sync_copy(den_v, den_sh.at[s])
    plsc.subcore_barrier()
    pltpu.sync_copy(den_sh.at[0, pl.ds(s * SPAN, SPAN)], racc_v)
    for r in range(1, NS):
        pltpu.sync_copy(den_sh.at[r, pl.ds(s * SPAN, SPAN)], rrow_v)

        def abody(i, carry):
            racc_v[pl.ds(i * L, L)] = (racc_v[pl.ds(i * L, L)]
                                       + rrow_v[pl.ds(i * L, L)])
            return carry
        lax.fori_loop(0, SPAN // L, abody, 0)

    @pl.when(c == 0)
    def _():
        pltpu.sync_copy(racc_v, den0_out.at[pl.ds(s * SPAN, SPAN)])

    @pl.when(c == 1)
    def _():
        pltpu.sync_copy(racc_v, den1_out.at[pl.ds(s * SPAN, SPAN)])


@functools.partial(
    pl.kernel,
    out_type=jax.ShapeDtypeStruct((NC, NPAD, H), jnp.float32),
    mesh=_mesh,
    scratch_types=[
        pltpu.VMEM_SHARED((NPAD, H), jnp.float32),
        pltpu.VMEM((PKW,), jnp.int32),
        pltpu.VMEM((PKW,), jnp.int32),
        pltpu.VMEM((CH2, H), jnp.float32),
        pltpu.VMEM((CH2, H), jnp.float32),
        pltpu.VMEM((SPAN,), jnp.float32),
        pltpu.VMEM((SPAN,), jnp.float32),
        pltpu.VMEM((D,), jnp.float32),
        pltpu.SemaphoreType.DMA,
        pltpu.SemaphoreType.DMA,
        pltpu.SemaphoreType.DMA,
        pltpu.SemaphoreType.DMA,
        pltpu.SemaphoreType.DMA,
        pltpu.SemaphoreType.DMA,
    ],
    compiler_params=pltpu.CompilerParams(needs_layout_passes=False, use_tc_tiling_on_sc=False),
)
def _pass2(xp_hbm, pk_hbm, den0_hbm, den1_hbm, bias_hbm, nout,
           acc_sh, pkA, pkB, rowsA, rowsB,
           den0_v, den1_v, bias_v, siA, siB, sgA, sgB, ssA, ssB):
    c = lax.axis_index("c")
    s = lax.axis_index("s")
    rows = pl.ds(s * SPAN, SPAN)
    HS = SPAN // 2

    def zrow(t, carry):
        for j in range(H // L):
            rowsA[t, pl.ds(j * L, L)] = jnp.zeros((L,), jnp.float32)
        return carry
    lax.fori_loop(0, HS, zrow, 0)
    for i in range(SPAN // HS):
        pltpu.sync_copy(rowsA.at[pl.ds(0, HS)],
                        acc_sh.at[pl.ds(s * SPAN + i * HS, HS)])
    plsc.subcore_barrier()

    def load_idx(k, pkX, sem):
        g = s * NCH + k
        pltpu.async_copy(pk_hbm.at[pl.ds(g * PKW, PKW)], pkX, sem)

    def wait_idx(pkX, sem):
        pltpu.make_async_copy(pk_hbm.at[pl.ds(0, PKW)], pkX, sem).wait()

    def gather(pkX, rowsX, sem):
        idx = pkX.at[pl.ds(0, CH2)]

        @pl.when(c == 0)
        def _():
            pltpu.async_copy(xp_hbm.at[0].at[idx], rowsX, sem)

        @pl.when(c == 1)
        def _():
            pltpu.async_copy(xp_hbm.at[1].at[idx], rowsX, sem)

    def wait_gather(pkX, rowsX, sem):
        pltpu.make_async_copy(
            xp_hbm.at[0].at[pkX.at[pl.ds(0, CH2)]], rowsX, sem).wait()

    def scatter(rowsX, pkX, sem):
        pltpu.async_copy(rowsX, acc_sh.at[pkX.at[pl.ds(CH2, CH2)]], sem,
                         add=True)

    def wait_scatter(rowsX, pkX, sem):
        pltpu.make_async_copy(
            rowsX, acc_sh.at[pkX.at[pl.ds(CH2, CH2)]], sem).wait()

    def scale(pkX, rowsX):
        def sbody(t, carry):
            e16 = plsc.bitcast(pkX[pl.ds(2 * CH2 + t * L, L)], jnp.float32)
            for u in range(L):
                r = t * L + u
                e = e16[u]
                for j in range(H // L):
                    rowsX[r, pl.ds(j * L, L)] = \
                        rowsX[r, pl.ds(j * L, L)] * e
            return carry
        lax.fori_loop(0, CH2 // L, sbody, 0)

    load_idx(0, pkA, siA)
    wait_idx(pkA, siA)
    gather(pkA, rowsA, sgA)
    load_idx(1, pkB, siB)

    def pair(p, carry):
        wait_idx(pkB, siB)

        @pl.when(p > 0)
        def _():
            wait_scatter(rowsB, pkB, ssB)
        gather(pkB, rowsB, sgB)
        wait_gather(pkA, rowsA, sgA)
        scale(pkA, rowsA)
        scatter(rowsA, pkA, ssA)
        wait_gather(pkB, rowsB, sgB)
        scale(pkB, rowsB)
        scatter(rowsB, pkB, ssB)

        @pl.when(p < NPAIR - 1)
        def _():
            wait_scatter(rowsA, pkA, ssA)
            load_idx(2 * p + 2, pkA, siA)
            wait_idx(pkA, siA)
            gather(pkA, rowsA, sgA)
            load_idx(2 * p + 3, pkB, siB)
        return carry
    lax.fori_loop(0, NPAIR, pair, 0)

    wait_scatter(rowsA, pkA, ssA)
    wait_scatter(rowsB, pkB, ssB)

    plsc.subcore_barrier()

    pltpu.sync_copy(den0_hbm.at[rows], den0_v)
    pltpu.sync_copy(den1_hbm.at[rows], den1_v)
    pltpu.sync_copy(bias_hbm, bias_v)
    bvs = [bias_v[pl.ds(c * H + j * L, L)] for j in range(H // L)]

    for i in range(SPAN // HS):
        half = pl.ds(s * SPAN + i * HS, HS)
        pltpu.sync_copy(acc_sh.at[half], rowsA.at[pl.ds(0, HS)])

        def fin(t, carry):
            d16 = (den0_v[pl.ds(i * HS + t * L, L)]
                   + den1_v[pl.ds(i * HS + t * L, L)]
                   + jnp.float32(1e-16))
            rc16 = jnp.float32(1.0) / d16
            for u in range(L):
                r = t * L + u
                rcp = rc16[u]
                for j in range(H // L):
                    rowsA[r, pl.ds(j * L, L)] = \
                        rowsA[r, pl.ds(j * L, L)] * rcp + bvs[j]
            return carry
        lax.fori_loop(0, HS // L, fin, 0)

        @pl.when(c == 0)
        def _():
            pltpu.sync_copy(rowsA.at[pl.ds(0, HS)], nout.at[0].at[half])

        @pl.when(c == 1)
        def _():
            pltpu.sync_copy(rowsA.at[pl.ds(0, HS)], nout.at[1].at[half])


def _fc_body(g_ref, w1_ref, b1_ref, w2_ref, b2_ref, o_ref, acc_ref):
    k = pl.program_id(0)

    @pl.when(k == 0)
    def _():
        acc_ref[...] = jnp.zeros_like(acc_ref)

    acc_ref[...] += lax.dot_general(g_ref[...], w1_ref[...],
                                    (((1,), (0,)), ((), ())),
                                    precision=lax.Precision.HIGHEST,
                                    preferred_element_type=jnp.float32)

    @pl.when(k == KN - 1)
    def _():
        h = jnp.maximum(acc_ref[...] + b1_ref[...], 0.0)
        o_ref[...] = lax.dot_general(h, w2_ref[...],
                                     (((1,), (0,)), ((), ())),
                                     precision=lax.Precision.HIGHEST,
                                     preferred_element_type=jnp.float32) \
            + b2_ref[...]


def _fc(g, W1, b1_row, W2, b2_row):
    nb = N // NPB
    return pl.pallas_call(
        _fc_body,
        grid=(KN,),
        in_specs=[
            pl.BlockSpec((nb, KBLK), lambda k: (0, k)),
            pl.BlockSpec((KBLK, HID), lambda k: (k, 0)),
            pl.BlockSpec((1, HID), lambda k: (0, 0)),
            pl.BlockSpec((HID, OUT_DIM), lambda k: (0, 0)),
            pl.BlockSpec((1, OUT_DIM), lambda k: (0, 0)),
        ],
        out_specs=pl.BlockSpec((nb, OUT_DIM), lambda k: (0, 0)),
        out_shape=jax.ShapeDtypeStruct((nb, OUT_DIM), jnp.float32),
        scratch_shapes=[pltpu.VMEM((nb, HID), jnp.float32)],
    )(g, W1, b1_row, W2, b2_row)


def kernel(x, edge_index, edge_attr, W, att_src, att_dst, W_e, att_edge,
           bias, W1, b1, W2, b2):
    x_pad = jnp.pad(x, ((0, NPAD - N), (0, 0)))
    att2 = jnp.stack([att_src, att_dst])
    xp, asd = _prep(x_pad, W, att2)
    ae = _ae(edge_attr.reshape(E // 8, D), W_e,
             att_edge.reshape(D, 1)).reshape(E)
    a_s_arr = asd[0]
    a_d_arr = asd[1]
    pk_sd = jnp.concatenate(
        [edge_index[0].reshape(NGRP, 1, CH2),
         edge_index[1].reshape(NGRP, 1, CH2),
         jnp.zeros((NGRP, 1, CH2), jnp.int32)], axis=1).reshape(E * 3)
    pk, den0, den1 = _pass1(a_s_arr, a_d_arr, pk_sd, ae)
    nout = _pass2(xp, pk, den0, den1, bias)
    g = jnp.concatenate([nout[0, :N], nout[1, :N]], axis=1)
    g = g.reshape(N // NPB, NPB * D)
    return _fc(g, W1, b1.reshape(1, HID), W2, b2.reshape(1, OUT_DIM))


# speedup vs baseline: 22.0799x; 1.5390x over previous
import functools

import jax
import jax.numpy as jnp
from jax import lax
from jax.experimental import pallas as pl
from jax.experimental.pallas import tpu as pltpu
from jax.experimental.pallas import tpu_sc as plsc

N = 10000
NPAD = 10240
E = 320000
D = 128
H = 64
D_EDGE = 16
NPB = 100
HID = 256
OUT_DIM = 64
NC = 2
NS = 16
L = 16
SPAN = NPAD // NS
EPT1 = E // (NC * NS)
CH1 = 2000
EPT2 = E // NS
CH2 = 400
NCH = EPT2 // CH2
NPAIR = NCH // 2
PKW = 3 * CH2
NGRP = E // CH2
GPT1 = CH1 // CH2
KBLK = 1280
KN = (NPB * D) // KBLK

_mesh = plsc.VectorSubcoreMesh(
    core_axis_name="c", subcore_axis_name="s", num_cores=NC, num_subcores=NS)


def _prep_body(x_ref, w_ref, att2_ref, xp_ref, asd_ref):
    xpb = lax.dot_general(x_ref[...], w_ref[...], (((1,), (0,)), ((), ())),
                          precision=lax.Precision.HIGHEST,
                          preferred_element_type=jnp.float32)
    xp_ref[0] = xpb[:, :H]
    xp_ref[1] = xpb[:, H:]
    asd_ref[...] = lax.dot_general(att2_ref[...], xpb,
                                   (((1,), (1,)), ((), ())),
                                   precision=lax.Precision.HIGHEST,
                                   preferred_element_type=jnp.float32)


def _prep(x_pad, W, att2):
    blk = 1280
    return pl.pallas_call(
        _prep_body,
        grid=(NPAD // blk,),
        in_specs=[
            pl.BlockSpec((blk, D), lambda i: (i, 0)),
            pl.BlockSpec((D, D), lambda i: (0, 0)),
            pl.BlockSpec((2, D), lambda i: (0, 0)),
        ],
        out_specs=[
            pl.BlockSpec((2, blk, H), lambda i: (0, i, 0)),
            pl.BlockSpec((2, blk), lambda i: (0, i)),
        ],
        out_shape=[
            jax.ShapeDtypeStruct((2, NPAD, H), jnp.float32),
            jax.ShapeDtypeStruct((2, NPAD), jnp.float32),
        ],
    )(x_pad, W, att2)


def _ae_body(attr_ref, we_ref, atte_ref, ae_ref):
    v = lax.dot_general(we_ref[...], atte_ref[...], (((1,), (0,)), ((), ())),
                        precision=lax.Precision.HIGHEST,
                        preferred_element_type=jnp.float32)
    vrep = jnp.concatenate([v] * 8, axis=0)
    vrep = jnp.broadcast_to(vrep, (D, 8))
    ki = lax.broadcasted_iota(jnp.int32, (D, 8), 0)
    ji = lax.broadcasted_iota(jnp.int32, (D, 8), 1)
    vdiag = jnp.where((ki // D_EDGE) == ji, vrep, 0.0)
    ae_ref[...] = lax.dot_general(attr_ref[...], vdiag,
                                  (((1,), (0,)), ((), ())),
                                  precision=lax.Precision.HIGHEST,
                                  preferred_element_type=jnp.float32)


def _ae(edge_attr8, W_e, att_edge_col):
    blk = 5000
    rows8 = E // 8
    return pl.pallas_call(
        _ae_body,
        grid=(rows8 // blk,),
        in_specs=[
            pl.BlockSpec((blk, D), lambda i: (i, 0)),
            pl.BlockSpec((D_EDGE, D), lambda i: (0, 0)),
            pl.BlockSpec((D, 1), lambda i: (0, 0)),
        ],
        out_specs=pl.BlockSpec((blk, 8), lambda i: (i, 0)),
        out_shape=jax.ShapeDtypeStruct((rows8, 8), jnp.float32),
    )(edge_attr8, W_e, att_edge_col)


@functools.partial(
    pl.kernel,
    out_type=[
        jax.ShapeDtypeStruct((E * 3,), jnp.int32),
        jax.ShapeDtypeStruct((NPAD,), jnp.float32),
        jax.ShapeDtypeStruct((NPAD,), jnp.float32),
    ],
    mesh=_mesh,
    scratch_types=[
        pltpu.VMEM((NPAD,), jnp.float32),
        pltpu.VMEM((NPAD,), jnp.float32),
        pltpu.VMEM((NPAD,), jnp.float32),
        pltpu.VMEM((GPT1 * PKW,), jnp.int32),
        pltpu.VMEM((CH1,), jnp.float32),
        pltpu.VMEM_SHARED((NS, NPAD), jnp.float32),
        pltpu.VMEM((SPAN,), jnp.float32),
        pltpu.VMEM((SPAN,), jnp.float32),
    ],
    compiler_params=pltpu.CompilerParams(needs_layout_passes=False, use_tc_tiling_on_sc=False),
)
def _pass1(a_s_hbm, a_d_hbm, pk_sd, ae, pk_out, den0_out, den1_out,
           a_s_v, a_d_v, den_v, pkv, ae_v,
           den_sh, racc_v, rrow_v):
    c = lax.axis_index("c")
    s = lax.axis_index("s")
    wid = c * NS + s
    pltpu.sync_copy(a_s_hbm, a_s_v)
    pltpu.sync_copy(a_d_hbm, a_d_v)

    def zbody(i, carry):
        den_v[pl.ds(i * L, L)] = jnp.zeros((L,), jnp.float32)
        return carry
    lax.fori_loop(0, NPAD // L, zbody, 0)

    for m in range(EPT1 // CH1):
        woff = (wid * EPT1 + m * CH1) * 3
        pltpu.sync_copy(pk_sd.at[pl.ds(woff, GPT1 * PKW)], pkv)
        pltpu.sync_copy(ae.at[pl.ds(wid * EPT1 + m * CH1, CH1)], ae_v)

        for grp in range(GPT1):
            goff = grp * PKW

            def ebody(j, carry):
                si = pkv[pl.ds(goff + j * L, L)]
                di = pkv[pl.ds(goff + CH2 + j * L, L)]
                av = (plsc.load_gather(a_s_v, [si])
                      + plsc.load_gather(a_d_v, [di])
                      + ae_v[pl.ds(grp * CH2 + j * L, L)])
                av = jnp.where(av >= 0.0, av, av * jnp.float32(0.2))
                ev = jnp.exp(av)
                pkv[pl.ds(goff + 2 * CH2 + j * L, L)] = \
                    plsc.bitcast(ev, jnp.int32)
                plsc.addupdate_scatter(den_v, [di], ev)
                return carry
            lax.fori_loop(0, CH2 // L, ebody, 0)
        pltpu.sync_copy(pkv, pk_out.at[pl.ds(woff, GPT1 * PKW)])

    pltpu.sync_copy(den_v, den_sh.at[s])
    plsc.subcore_barrier()
    pltpu.sync_copy(den_sh.at[0, pl.ds(s * SPAN, SPAN)], racc_v)
    for r in range(1, NS):
        pltpu.sync_copy(den_sh.at[r, pl.ds(s * SPAN, SPAN)], rrow_v)

        def abody(i, carry):
            racc_v[pl.ds(i * L, L)] = (racc_v[pl.ds(i * L, L)]
                                       + rrow_v[pl.ds(i * L, L)])
            return carry
        lax.fori_loop(0, SPAN // L, abody, 0)

    @pl.when(c == 0)
    def _():
        pltpu.sync_copy(racc_v, den0_out.at[pl.ds(s * SPAN, SPAN)])

    @pl.when(c == 1)
    def _():
        pltpu.sync_copy(racc_v, den1_out.at[pl.ds(s * SPAN, SPAN)])


@functools.partial(
    pl.kernel,
    out_type=jax.ShapeDtypeStruct((NC, NPAD, H), jnp.float32),
    mesh=_mesh,
    scratch_types=[
        pltpu.VMEM_SHARED((NPAD, H), jnp.float32),
        pltpu.VMEM((PKW,), jnp.int32),
        pltpu.VMEM((PKW,), jnp.int32),
        pltpu.VMEM((CH2, H), jnp.float32),
        pltpu.VMEM((CH2, H), jnp.float32),
        pltpu.VMEM((CH2,), jnp.int32),
        pltpu.VMEM((CH2,), jnp.int32),
        pltpu.VMEM((SPAN,), jnp.float32),
        pltpu.VMEM((SPAN,), jnp.float32),
        pltpu.VMEM((D,), jnp.float32),
        pltpu.SemaphoreType.DMA,
        pltpu.SemaphoreType.DMA,
        pltpu.SemaphoreType.DMA,
        pltpu.SemaphoreType.DMA,
        pltpu.SemaphoreType.DMA,
        pltpu.SemaphoreType.DMA,
    ],
    compiler_params=pltpu.CompilerParams(needs_layout_passes=False, use_tc_tiling_on_sc=False),
)
def _pass2(xp_hbm, pk_hbm, den0_hbm, den1_hbm, bias_hbm, nout,
           acc_sh, pkA, pkB, rowsA, rowsB, dstA, dstB,
           den0_v, den1_v, bias_v, siA, siB, sgA, sgB, ssA, ssB):
    c = lax.axis_index("c")
    s = lax.axis_index("s")
    rows = pl.ds(s * SPAN, SPAN)
    HS = SPAN // 2

    def zrow(t, carry):
        for j in range(H // L):
            rowsA[t, pl.ds(j * L, L)] = jnp.zeros((L,), jnp.float32)
        return carry
    lax.fori_loop(0, HS, zrow, 0)
    for i in range(SPAN // HS):
        pltpu.sync_copy(rowsA.at[pl.ds(0, HS)],
                        acc_sh.at[pl.ds(s * SPAN + i * HS, HS)])
    plsc.subcore_barrier()

    def load_idx(k, pkX, sem):
        g = s * NCH + k
        pltpu.async_copy(pk_hbm.at[pl.ds(g * PKW, PKW)], pkX, sem)

    def wait_idx(pkX, sem):
        pltpu.make_async_copy(pk_hbm.at[pl.ds(0, PKW)], pkX, sem).wait()

    def gather(pkX, rowsX, sem):
        idx = pkX.at[pl.ds(0, CH2)]

        @pl.when(c == 0)
        def _():
            pltpu.async_copy(xp_hbm.at[0].at[idx], rowsX, sem)

        @pl.when(c == 1)
        def _():
            pltpu.async_copy(xp_hbm.at[1].at[idx], rowsX, sem)

    def wait_gather(pkX, rowsX, sem):
        pltpu.make_async_copy(
            xp_hbm.at[0].at[pkX.at[pl.ds(0, CH2)]], rowsX, sem).wait()

    def copy_dst(pkX, dstX):
        for t0 in range(0, CH2 // L, 5):
            vals = [pkX[pl.ds(CH2 + (t0 + t) * L, L)] for t in range(5)]
            for t in range(5):
                dstX[pl.ds((t0 + t) * L, L)] = vals[t]

    def scatter(rowsX, dstX, sem):
        pltpu.async_copy(rowsX, acc_sh.at[dstX], sem, add=True)

    def wait_scatter(rowsX, dstX, sem):
        pltpu.make_async_copy(rowsX, acc_sh.at[dstX], sem).wait()

    def scale(pkX, rowsX):
        GR = 8

        def sbody(t, carry):
            e16 = plsc.bitcast(pkX[pl.ds(2 * CH2 + t * L, L)], jnp.float32)
            for u0 in range(0, L, GR):
                prods = []
                for u in range(u0, u0 + GR):
                    r = t * L + u
                    e = e16[u]
                    prods.append([rowsX[r, pl.ds(j * L, L)] * e
                                  for j in range(H // L)])
                for g in range(GR):
                    r = t * L + u0 + g
                    for j in range(H // L):
                        rowsX[r, pl.ds(j * L, L)] = prods[g][j]
            return carry
        lax.fori_loop(0, CH2 // L, sbody, 0)

    load_idx(0, pkA, siA)
    wait_idx(pkA, siA)
    gather(pkA, rowsA, sgA)
    load_idx(1, pkB, siB)

    def pair(p, carry):
        wait_idx(pkB, siB)

        @pl.when(p > 0)
        def _():
            wait_scatter(rowsB, dstB, ssB)
        gather(pkB, rowsB, sgB)
        wait_gather(pkA, rowsA, sgA)
        scale(pkA, rowsA)
        copy_dst(pkA, dstA)
        scatter(rowsA, dstA, ssA)
        wait_gather(pkB, rowsB, sgB)
        scale(pkB, rowsB)
        copy_dst(pkB, dstB)
        scatter(rowsB, dstB, ssB)

        @pl.when(p < NPAIR - 1)
        def _():
            wait_scatter(rowsA, dstA, ssA)
            load_idx(2 * p + 2, pkA, siA)
            wait_idx(pkA, siA)
            gather(pkA, rowsA, sgA)
            load_idx(2 * p + 3, pkB, siB)
        return carry
    lax.fori_loop(0, NPAIR, pair, 0)

    wait_scatter(rowsA, dstA, ssA)
    wait_scatter(rowsB, dstB, ssB)

    plsc.subcore_barrier()

    pltpu.sync_copy(den0_hbm.at[rows], den0_v)
    pltpu.sync_copy(den1_hbm.at[rows], den1_v)
    pltpu.sync_copy(bias_hbm, bias_v)
    bvs = [bias_v[pl.ds(c * H + j * L, L)] for j in range(H // L)]

    for i in range(SPAN // HS):
        half = pl.ds(s * SPAN + i * HS, HS)
        pltpu.sync_copy(acc_sh.at[half], rowsA.at[pl.ds(0, HS)])

        def fin(t, carry):
            d16 = (den0_v[pl.ds(i * HS + t * L, L)]
                   + den1_v[pl.ds(i * HS + t * L, L)]
                   + jnp.float32(1e-16))
            rc16 = jnp.float32(1.0) / d16
            for u0 in range(0, L, 8):
                prods = []
                for u in range(u0, u0 + 8):
                    r = t * L + u
                    rcp = rc16[u]
                    prods.append([rowsA[r, pl.ds(j * L, L)] * rcp + bvs[j]
                                  for j in range(H // L)])
                for g in range(8):
                    r = t * L + u0 + g
                    for j in range(H // L):
                        rowsA[r, pl.ds(j * L, L)] = prods[g][j]
            return carry
        lax.fori_loop(0, HS // L, fin, 0)

        @pl.when(c == 0)
        def _():
            pltpu.sync_copy(rowsA.at[pl.ds(0, HS)], nout.at[0].at[half])

        @pl.when(c == 1)
        def _():
            pltpu.sync_copy(rowsA.at[pl.ds(0, HS)], nout.at[1].at[half])


def _fc_body(g_ref, w1_ref, b1_ref, w2_ref, b2_ref, o_ref, acc_ref):
    k = pl.program_id(0)

    @pl.when(k == 0)
    def _():
        acc_ref[...] = jnp.zeros_like(acc_ref)

    acc_ref[...] += lax.dot_general(g_ref[...], w1_ref[...],
                                    (((1,), (0,)), ((), ())),
                                    precision=lax.Precision.HIGHEST,
                                    preferred_element_type=jnp.float32)

    @pl.when(k == KN - 1)
    def _():
        h = jnp.maximum(acc_ref[...] + b1_ref[...], 0.0)
        o_ref[...] = lax.dot_general(h, w2_ref[...],
                                     (((1,), (0,)), ((), ())),
                                     precision=lax.Precision.HIGHEST,
                                     preferred_element_type=jnp.float32) \
            + b2_ref[...]


def _fc(g, W1, b1_row, W2, b2_row):
    nb = N // NPB
    return pl.pallas_call(
        _fc_body,
        grid=(KN,),
        in_specs=[
            pl.BlockSpec((nb, KBLK), lambda k: (0, k)),
            pl.BlockSpec((KBLK, HID), lambda k: (k, 0)),
            pl.BlockSpec((1, HID), lambda k: (0, 0)),
            pl.BlockSpec((HID, OUT_DIM), lambda k: (0, 0)),
            pl.BlockSpec((1, OUT_DIM), lambda k: (0, 0)),
        ],
        out_specs=pl.BlockSpec((nb, OUT_DIM), lambda k: (0, 0)),
        out_shape=jax.ShapeDtypeStruct((nb, OUT_DIM), jnp.float32),
        scratch_shapes=[pltpu.VMEM((nb, HID), jnp.float32)],
    )(g, W1, b1_row, W2, b2_row)


def kernel(x, edge_index, edge_attr, W, att_src, att_dst, W_e, att_edge,
           bias, W1, b1, W2, b2):
    x_pad = jnp.pad(x, ((0, NPAD - N), (0, 0)))
    att2 = jnp.stack([att_src, att_dst])
    xp, asd = _prep(x_pad, W, att2)
    ae = _ae(edge_attr.reshape(E // 8, D), W_e,
             att_edge.reshape(D, 1)).reshape(E)
    a_s_arr = asd[0]
    a_d_arr = asd[1]
    pk_sd = jnp.concatenate(
        [edge_index[0].reshape(NGRP, 1, CH2),
         edge_index[1].reshape(NGRP, 1, CH2),
         jnp.zeros((NGRP, 1, CH2), jnp.int32)], axis=1).reshape(E * 3)
    pk, den0, den1 = _pass1(a_s_arr, a_d_arr, pk_sd, ae)
    nout = _pass2(xp, pk, den0, den1, bias)
    g = jnp.concatenate([nout[0, :N], nout[1, :N]], axis=1)
    g = g.reshape(N // NPB, NPB * D)
    return _fc(g, W1, b1.reshape(1, HID), W2, b2.reshape(1, OUT_DIM))


# speedup vs baseline: 23.2929x; 1.0549x over previous
import functools

import jax
import jax.numpy as jnp
from jax import lax
from jax.experimental import pallas as pl
from jax.experimental.pallas import tpu as pltpu
from jax.experimental.pallas import tpu_sc as plsc

N = 10000
NPAD = 10240
E = 320000
D = 128
H = 64
D_EDGE = 16
NPB = 100
HID = 256
OUT_DIM = 64
NC = 2
NS = 16
L = 16
SPAN = NPAD // NS
EPT1 = E // (NC * NS)
CH1 = 2000
EPT2 = E // NS
CH2 = 400
NCH = EPT2 // CH2
NPAIR = NCH // 2
PKW = 3 * CH2
NGRP = E // CH2
GPT1 = CH1 // CH2
KBLK = 1280
KN = (NPB * D) // KBLK

_mesh = plsc.VectorSubcoreMesh(
    core_axis_name="c", subcore_axis_name="s", num_cores=NC, num_subcores=NS)


def _prep_body(x_ref, w_ref, att2_ref, attr_ref, we_ref, atte_ref,
               xp_ref, asd_ref, ae_ref):
    xpb = lax.dot_general(x_ref[...], w_ref[...], (((1,), (0,)), ((), ())),
                          precision=lax.Precision.HIGHEST,
                          preferred_element_type=jnp.float32)
    xp_ref[0] = xpb[:, :H]
    xp_ref[1] = xpb[:, H:]
    ws2 = lax.dot_general(w_ref[...], att2_ref[...],
                          (((1,), (1,)), ((), ())),
                          precision=lax.Precision.HIGHEST,
                          preferred_element_type=jnp.float32)
    asd_ref[...] = lax.dot_general(x_ref[...], ws2,
                                   (((1,), (0,)), ((), ())),
                                   precision=lax.Precision.HIGHEST,
                                   preferred_element_type=jnp.float32)
    v = lax.dot_general(we_ref[...], atte_ref[...], (((1,), (0,)), ((), ())),
                        precision=lax.Precision.HIGHEST,
                        preferred_element_type=jnp.float32)
    vrep = jnp.broadcast_to(jnp.concatenate([v] * 8, axis=0), (D, 8))
    ki = lax.broadcasted_iota(jnp.int32, (D, 8), 0)
    ji = lax.broadcasted_iota(jnp.int32, (D, 8), 1)
    vdiag = jnp.where((ki // D_EDGE) == ji, vrep, 0.0)
    ae_ref[...] = lax.dot_general(attr_ref[...], vdiag,
                                  (((1,), (0,)), ((), ())),
                                  precision=lax.Precision.HIGHEST,
                                  preferred_element_type=jnp.float32)


def _prep(x, W, att2, edge_attr8, W_e, att_edge_col):
    blk = 1000
    eblk = (E // 8) // (N // blk)
    return pl.pallas_call(
        _prep_body,
        grid=(N // blk,),
        in_specs=[
            pl.BlockSpec((blk, D), lambda i: (i, 0)),
            pl.BlockSpec((D, D), lambda i: (0, 0)),
            pl.BlockSpec((2, D), lambda i: (0, 0)),
            pl.BlockSpec((eblk, D), lambda i: (i, 0)),
            pl.BlockSpec((D_EDGE, D), lambda i: (0, 0)),
            pl.BlockSpec((D, 1), lambda i: (0, 0)),
        ],
        out_specs=[
            pl.BlockSpec((2, blk, H), lambda i: (0, i, 0)),
            pl.BlockSpec((blk, 2), lambda i: (i, 0)),
            pl.BlockSpec((eblk, 8), lambda i: (i, 0)),
        ],
        out_shape=[
            jax.ShapeDtypeStruct((2, N, H), jnp.float32),
            jax.ShapeDtypeStruct((N, 2), jnp.float32),
            jax.ShapeDtypeStruct((E // 8, 8), jnp.float32),
        ],
    )(x, W, att2, edge_attr8, W_e, att_edge_col)


@functools.partial(
    pl.kernel,
    out_type=[
        jax.ShapeDtypeStruct((E * 3,), jnp.int32),
        jax.ShapeDtypeStruct((NPAD,), jnp.float32),
        jax.ShapeDtypeStruct((NPAD,), jnp.float32),
    ],
    mesh=_mesh,
    scratch_types=[
        pltpu.VMEM((N,), jnp.float32),
        pltpu.VMEM((N,), jnp.float32),
        pltpu.VMEM((NPAD,), jnp.float32),
        pltpu.VMEM((GPT1 * PKW,), jnp.int32),
        pltpu.VMEM((CH1,), jnp.int32),
        pltpu.VMEM((CH1,), jnp.int32),
        pltpu.VMEM((CH1,), jnp.float32),
        pltpu.VMEM_SHARED((NS, NPAD), jnp.float32),
        pltpu.VMEM((SPAN,), jnp.float32),
        pltpu.VMEM((SPAN,), jnp.float32),
    ],
    compiler_params=pltpu.CompilerParams(needs_layout_passes=False, use_tc_tiling_on_sc=False),
)
def _pass1(a_s_hbm, a_d_hbm, srcE, dstE, ae, pk_out, den0_out, den1_out,
           a_s_v, a_d_v, den_v, pkv, src_v, dst_v, ae_v,
           den_sh, racc_v, rrow_v):
    c = lax.axis_index("c")
    s = lax.axis_index("s")
    wid = c * NS + s
    pltpu.sync_copy(a_s_hbm, a_s_v)
    pltpu.sync_copy(a_d_hbm, a_d_v)

    def zbody(i, carry):
        den_v[pl.ds(i * L, L)] = jnp.zeros((L,), jnp.float32)
        return carry
    lax.fori_loop(0, NPAD // L, zbody, 0)

    for m in range(EPT1 // CH1):
        ebase = wid * EPT1 + m * CH1
        pltpu.sync_copy(srcE.at[pl.ds(ebase, CH1)], src_v)
        pltpu.sync_copy(dstE.at[pl.ds(ebase, CH1)], dst_v)
        pltpu.sync_copy(ae.at[pl.ds(ebase, CH1)], ae_v)

        for grp in range(GPT1):
            goff = grp * PKW

            def ebody(j, carry):
                si = src_v[pl.ds(grp * CH2 + j * L, L)]
                di = dst_v[pl.ds(grp * CH2 + j * L, L)]
                av = (plsc.load_gather(a_s_v, [si])
                      + plsc.load_gather(a_d_v, [di])
                      + ae_v[pl.ds(grp * CH2 + j * L, L)])
                av = jnp.where(av >= 0.0, av, av * jnp.float32(0.2))
                ev = jnp.exp(av)
                pkv[pl.ds(goff + j * L, L)] = si
                pkv[pl.ds(goff + CH2 + j * L, L)] = di
                pkv[pl.ds(goff + 2 * CH2 + j * L, L)] = \
                    plsc.bitcast(ev, jnp.int32)
                plsc.addupdate_scatter(den_v, [di], ev)
                return carry
            lax.fori_loop(0, CH2 // L, ebody, 0)
        pltpu.sync_copy(pkv, pk_out.at[pl.ds(ebase * 3, GPT1 * PKW)])

    pltpu.sync_copy(den_v, den_sh.at[s])
    plsc.subcore_barrier()
    pltpu.sync_copy(den_sh.at[0, pl.ds(s * SPAN, SPAN)], racc_v)
    for r in range(1, NS):
        pltpu.sync_copy(den_sh.at[r, pl.ds(s * SPAN, SPAN)], rrow_v)

        def abody(i, carry):
            racc_v[pl.ds(i * L, L)] = (racc_v[pl.ds(i * L, L)]
                                       + rrow_v[pl.ds(i * L, L)])
            return carry
        lax.fori_loop(0, SPAN // L, abody, 0)

    @pl.when(c == 0)
    def _():
        pltpu.sync_copy(racc_v, den0_out.at[pl.ds(s * SPAN, SPAN)])

    @pl.when(c == 1)
    def _():
        pltpu.sync_copy(racc_v, den1_out.at[pl.ds(s * SPAN, SPAN)])


@functools.partial(
    pl.kernel,
    out_type=jax.ShapeDtypeStruct((NC, NPAD, H), jnp.float32),
    mesh=_mesh,
    scratch_types=[
        pltpu.VMEM_SHARED((NPAD, H), jnp.float32),
        pltpu.VMEM((PKW,), jnp.int32),
        pltpu.VMEM((PKW,), jnp.int32),
        pltpu.VMEM((CH2, H), jnp.float32),
        pltpu.VMEM((CH2, H), jnp.float32),
        pltpu.VMEM((CH2,), jnp.int32),
        pltpu.VMEM((CH2,), jnp.int32),
        pltpu.VMEM((SPAN,), jnp.float32),
        pltpu.VMEM((SPAN,), jnp.float32),
        pltpu.VMEM((D,), jnp.float32),
        pltpu.SemaphoreType.DMA,
        pltpu.SemaphoreType.DMA,
        pltpu.SemaphoreType.DMA,
        pltpu.SemaphoreType.DMA,
        pltpu.SemaphoreType.DMA,
        pltpu.SemaphoreType.DMA,
    ],
    compiler_params=pltpu.CompilerParams(needs_layout_passes=False, use_tc_tiling_on_sc=False),
)
def _pass2(xp_hbm, pk_hbm, den0_hbm, den1_hbm, bias_hbm, nout,
           acc_sh, pkA, pkB, rowsA, rowsB, dstA, dstB,
           den0_v, den1_v, bias_v, siA, siB, sgA, sgB, ssA, ssB):
    c = lax.axis_index("c")
    s = lax.axis_index("s")
    rows = pl.ds(s * SPAN, SPAN)
    HS = SPAN // 2

    def zrow(t, carry):
        for j in range(H // L):
            rowsA[t, pl.ds(j * L, L)] = jnp.zeros((L,), jnp.float32)
        return carry
    lax.fori_loop(0, HS, zrow, 0)
    for i in range(SPAN // HS):
        pltpu.sync_copy(rowsA.at[pl.ds(0, HS)],
                        acc_sh.at[pl.ds(s * SPAN + i * HS, HS)])
    plsc.subcore_barrier()

    def load_idx(k, pkX, sem):
        g = s * NCH + k
        pltpu.async_copy(pk_hbm.at[pl.ds(g * PKW, PKW)], pkX, sem)

    def wait_idx(pkX, sem):
        pltpu.make_async_copy(pk_hbm.at[pl.ds(0, PKW)], pkX, sem).wait()

    def gather(pkX, rowsX, sem):
        idx = pkX.at[pl.ds(0, CH2)]

        @pl.when(c == 0)
        def _():
            pltpu.async_copy(xp_hbm.at[0].at[idx], rowsX, sem)

        @pl.when(c == 1)
        def _():
            pltpu.async_copy(xp_hbm.at[1].at[idx], rowsX, sem)

    def wait_gather(pkX, rowsX, sem):
        pltpu.make_async_copy(
            xp_hbm.at[0].at[pkX.at[pl.ds(0, CH2)]], rowsX, sem).wait()

    def copy_dst(pkX, dstX):
        for t0 in range(0, CH2 // L, 5):
            vals = [pkX[pl.ds(CH2 + (t0 + t) * L, L)] for t in range(5)]
            for t in range(5):
                dstX[pl.ds((t0 + t) * L, L)] = vals[t]

    def scatter(rowsX, dstX, sem):
        pltpu.async_copy(rowsX, acc_sh.at[dstX], sem, add=True)

    def wait_scatter(rowsX, dstX, sem):
        pltpu.make_async_copy(rowsX, acc_sh.at[dstX], sem).wait()

    def scale(pkX, rowsX):
        GR = 8

        def sbody(t, carry):
            e16 = plsc.bitcast(pkX[pl.ds(2 * CH2 + t * L, L)], jnp.float32)
            for u0 in range(0, L, GR):
                prods = []
                for u in range(u0, u0 + GR):
                    r = t * L + u
                    e = e16[u]
                    prods.append([rowsX[r, pl.ds(j * L, L)] * e
                                  for j in range(H // L)])
                for g in range(GR):
                    r = t * L + u0 + g
                    for j in range(H // L):
                        rowsX[r, pl.ds(j * L, L)] = prods[g][j]
            return carry
        lax.fori_loop(0, CH2 // L, sbody, 0)

    load_idx(0, pkA, siA)
    wait_idx(pkA, siA)
    gather(pkA, rowsA, sgA)
    load_idx(1, pkB, siB)

    def pair(p, carry):
        wait_idx(pkB, siB)

        @pl.when(p > 0)
        def _():
            wait_scatter(rowsB, dstB, ssB)
        gather(pkB, rowsB, sgB)
        wait_gather(pkA, rowsA, sgA)
        scale(pkA, rowsA)
        copy_dst(pkA, dstA)
        scatter(rowsA, dstA, ssA)
        wait_gather(pkB, rowsB, sgB)
        scale(pkB, rowsB)
        copy_dst(pkB, dstB)
        scatter(rowsB, dstB, ssB)

        @pl.when(p < NPAIR - 1)
        def _():
            wait_scatter(rowsA, dstA, ssA)
            load_idx(2 * p + 2, pkA, siA)
            wait_idx(pkA, siA)
            gather(pkA, rowsA, sgA)
            load_idx(2 * p + 3, pkB, siB)
        return carry
    lax.fori_loop(0, NPAIR, pair, 0)

    wait_scatter(rowsA, dstA, ssA)
    wait_scatter(rowsB, dstB, ssB)

    plsc.subcore_barrier()

    pltpu.sync_copy(den0_hbm.at[rows], den0_v)
    pltpu.sync_copy(den1_hbm.at[rows], den1_v)
    pltpu.sync_copy(bias_hbm, bias_v)
    bvs = [bias_v[pl.ds(c * H + j * L, L)] for j in range(H // L)]

    for i in range(SPAN // HS):
        half = pl.ds(s * SPAN + i * HS, HS)
        pltpu.sync_copy(acc_sh.at[half], rowsA.at[pl.ds(0, HS)])

        def fin(t, carry):
            d16 = (den0_v[pl.ds(i * HS + t * L, L)]
                   + den1_v[pl.ds(i * HS + t * L, L)]
                   + jnp.float32(1e-16))
            rc16 = jnp.float32(1.0) / d16
            for u0 in range(0, L, 8):
                prods = []
                for u in range(u0, u0 + 8):
                    r = t * L + u
                    rcp = rc16[u]
                    prods.append([rowsA[r, pl.ds(j * L, L)] * rcp + bvs[j]
                                  for j in range(H // L)])
                for g in range(8):
                    r = t * L + u0 + g
                    for j in range(H // L):
                        rowsA[r, pl.ds(j * L, L)] = prods[g][j]
            return carry
        lax.fori_loop(0, HS // L, fin, 0)

        @pl.when(c == 0)
        def _():
            pltpu.sync_copy(rowsA.at[pl.ds(0, HS)], nout.at[0].at[half])

        @pl.when(c == 1)
        def _():
            pltpu.sync_copy(rowsA.at[pl.ds(0, HS)], nout.at[1].at[half])


def _fc_body(g_ref, w1_ref, b1_ref, w2_ref, b2_ref, o_ref, acc_ref):
    k = pl.program_id(0)

    @pl.when(k == 0)
    def _():
        acc_ref[...] = jnp.zeros_like(acc_ref)

    acc_ref[...] += lax.dot_general(g_ref[...], w1_ref[...],
                                    (((1,), (0,)), ((), ())),
                                    precision=lax.Precision.HIGHEST,
                                    preferred_element_type=jnp.float32)

    @pl.when(k == KN - 1)
    def _():
        h = jnp.maximum(acc_ref[...] + b1_ref[...], 0.0)
        o_ref[...] = lax.dot_general(h, w2_ref[...],
                                     (((1,), (0,)), ((), ())),
                                     precision=lax.Precision.HIGHEST,
                                     preferred_element_type=jnp.float32) \
            + b2_ref[...]


def _fc(g, W1, b1_row, W2, b2_row):
    nb = N // NPB
    return pl.pallas_call(
        _fc_body,
        grid=(KN,),
        in_specs=[
            pl.BlockSpec((nb, KBLK), lambda k: (0, k)),
            pl.BlockSpec((KBLK, HID), lambda k: (k, 0)),
            pl.BlockSpec((1, HID), lambda k: (0, 0)),
            pl.BlockSpec((HID, OUT_DIM), lambda k: (0, 0)),
            pl.BlockSpec((1, OUT_DIM), lambda k: (0, 0)),
        ],
        out_specs=pl.BlockSpec((nb, OUT_DIM), lambda k: (0, 0)),
        out_shape=jax.ShapeDtypeStruct((nb, OUT_DIM), jnp.float32),
        scratch_shapes=[pltpu.VMEM((nb, HID), jnp.float32)],
    )(g, W1, b1_row, W2, b2_row)


def kernel(x, edge_index, edge_attr, W, att_src, att_dst, W_e, att_edge,
           bias, W1, b1, W2, b2):
    att2 = jnp.stack([att_src, att_dst])
    xp, asd, ae8 = _prep(x, W, att2, edge_attr.reshape(E // 8, D),
                         W_e, att_edge.reshape(D, 1))
    ae = ae8.reshape(E)
    pk, den0, den1 = _pass1(asd[:, 0], asd[:, 1],
                            edge_index[0], edge_index[1], ae)
    nout = _pass2(xp, pk, den0, den1, bias)
    g = jnp.concatenate([nout[0, :N], nout[1, :N]], axis=1)
    g = g.reshape(N // NPB, NPB * D)
    return _fc(g, W1, b1.reshape(1, HID), W2, b2.reshape(1, OUT_DIM))
